# baseline (speedup 1.0000x reference)
import functools

import jax
import jax.numpy as jnp
from jax import lax
from jax.experimental import pallas as pl
from jax.experimental.pallas import tpu as pltpu
from jax.experimental.pallas import tpu_sc as plsc

B = 4096
L = 200
D = 64
NC = 2
NS = 16
NW = NC * NS
BPW = B // NW
NCH = D // 16
NG = L // 16
TAIL = L - 16 * NG
UN = 8

_mesh = plsc.VectorSubcoreMesh(core_axis_name="c", subcore_axis_name="s")


@functools.partial(
    pl.kernel,
    mesh=_mesh,
    out_type=jax.ShapeDtypeStruct((B // 2, 2 * D), jnp.float32),
    scratch_types=[
        pltpu.VMEM((BPW, L), jnp.int32),
        pltpu.VMEM((L, D), jnp.float32),
        pltpu.VMEM((L, D), jnp.float32),
        pltpu.VMEM((BPW // 2, 2 * D), jnp.float32),
        pltpu.SemaphoreType.DMA,
        pltpu.SemaphoreType.DMA,
    ],
)
def _emb_pool(x_hbm, table_hbm, out_hbm, idx_v, rows_a, rows_b, out_v,
              sem_a, sem_b):
    wid = lax.axis_index("s") * NC + lax.axis_index("c")
    pltpu.sync_copy(x_hbm.at[pl.ds(wid * BPW, BPW)], idx_v)

    def issue(elt, buf, sem):
        def issue_group(g, carry):
            for h in range(2):
                base = 32 * g + 16 * h
                q16 = idx_v[elt, pl.ds(base, 16)]
                for k in range(16):
                    pltpu.make_async_copy(
                        table_hbm.at[pl.ds(q16[k], 1)],
                        buf.at[pl.ds(base + k, 1)],
                        sem,
                    ).start()
            return carry

        lax.fori_loop(0, NG // 2, issue_group, 0)
        q16 = idx_v[elt, pl.ds(L - 16, 16)]
        for k in range(16 - TAIL, 16):
            pltpu.make_async_copy(
                table_hbm.at[pl.ds(q16[k], 1)],
                buf.at[pl.ds(L - 16 + k, 1)],
                sem,
            ).start()

    def drain(buf, sem):
        pltpu.make_async_copy(table_hbm.at[pl.ds(0, L)], buf, sem).wait()

    def accumulate(buf, row, half):
        def acc_body(j, accs):
            r = j * UN
            new = list(accs)
            for k in range(UN):
                for c in range(NCH):
                    new[c] = new[c] + buf[r + k, pl.ds(c * 16, 16)]
            return tuple(new)

        accs = lax.fori_loop(
            0, L // UN, acc_body,
            tuple(jnp.zeros((16,), jnp.float32) for _ in range(NCH)),
        )
        for c in range(NCH):
            out_v[row, pl.ds(half * D + c * 16, 16)] = accs[c] * (1.0 / L)

    issue(0, rows_a, sem_a)

    def outer(i, carry):
        b0 = 2 * i
        issue(b0 + 1, rows_b, sem_b)
        drain(rows_a, sem_a)
        accumulate(rows_a, i, 0)
        issue(jnp.minimum(b0 + 2, BPW - 1), rows_a, sem_a)
        drain(rows_b, sem_b)
        accumulate(rows_b, i, 1)
        return carry

    lax.fori_loop(0, BPW // 2, outer, 0)
    drain(rows_a, sem_a)
    pltpu.sync_copy(out_v, out_hbm.at[pl.ds(wid * (BPW // 2), BPW // 2)])


def kernel(x, table):
    return _emb_pool(x.astype(jnp.int32), table).reshape(B, D)
